# Initial kernel scaffold; baseline (speedup 1.0000x reference)
import jax
import jax.numpy as jnp
from jax import lax
from jax.experimental import pallas as pl
from jax.experimental.pallas import tpu as pltpu
from jax.experimental.pallas import tpu_sc as plsc

N_TOKENS = 32768
B_SEGS = 16
N_WORKERS = 16
CHUNK = N_TOKENS // N_WORKERS
LANES = 16


def _jagged_argmax_body(values_hbm, ps_hbm, out_hbm,
                        vals_v, ps_v, pm_v, pi_v, mm_v, mi_v, ans_v,
                        sh_max, sh_idx):
    cid = lax.axis_index("c")
    sid = lax.axis_index("s")

    @pl.when(cid == 0)
    def _():
        pltpu.sync_copy(ps_hbm, ps_v)
        lo = sid * CHUNK
        pltpu.sync_copy(values_hbm.at[pl.ds(lo, CHUNK)], vals_v)

        lane = lax.iota(jnp.int32, LANES)
        ninf = jnp.full((LANES,), -jnp.inf, jnp.float32)
        sent = jnp.full((LANES,), N_TOKENS, jnp.int32)

        for k in range(B_SEGS):
            start = ps_v[k - 1] if k > 0 else jnp.int32(0)
            end = ps_v[k]
            a = jnp.maximum(start, lo)
            b = jnp.minimum(end, lo + CHUNK)
            va = lax.shift_right_logical(a, 4)
            vb = lax.shift_right_logical(b + (LANES - 1), 4)

            def body(j, carry, a=a, b=b, lo=lo):
                cm, ci = carry
                off = j * LANES
                v = vals_v[pl.ds(off - lo, LANES)]
                pos = off + lane
                valid = (pos >= a) & (pos < b)
                vm = jnp.where(valid, v, ninf)
                upd = vm > cm
                return jnp.where(upd, vm, cm), jnp.where(upd, pos, ci)

            cm, ci = lax.fori_loop(va, vb, body, (ninf, sent))
            m = jnp.max(cm)
            cand = jnp.where(cm == m, ci, sent)
            pm_v[k] = m
            pi_v[k] = jnp.min(cand)

        pltpu.sync_copy(pm_v, sh_max.at[pl.ds(sid * B_SEGS, B_SEGS)])
        pltpu.sync_copy(pi_v, sh_idx.at[pl.ds(sid * B_SEGS, B_SEGS)])
        plsc.subcore_barrier()

        @pl.when(sid == 0)
        def _():
            pltpu.sync_copy(sh_max, mm_v)
            pltpu.sync_copy(sh_idx, mi_v)
            cm = mm_v[pl.ds(0, B_SEGS)]
            ci = mi_v[pl.ds(0, B_SEGS)]
            for w in range(1, N_WORKERS):
                wm = mm_v[pl.ds(w * B_SEGS, B_SEGS)]
                wi = mi_v[pl.ds(w * B_SEGS, B_SEGS)]
                better = (wm > cm) | ((wm == cm) & (wi < ci))
                cm = jnp.where(better, wm, cm)
                ci = jnp.where(better, wi, ci)
            ans_v[...] = jnp.where(ci >= N_TOKENS, -1, ci)
            pltpu.sync_copy(ans_v, out_hbm)


@jax.jit
def kernel(values, prefix_sum):
    ps32 = prefix_sum.astype(jnp.int32)
    mesh = plsc.VectorSubcoreMesh(
        core_axis_name="c", subcore_axis_name="s", num_cores=1
    )
    out = pl.kernel(
        _jagged_argmax_body,
        out_type=jax.ShapeDtypeStruct((B_SEGS,), jnp.int32),
        mesh=mesh,
        scratch_types=[
            pltpu.VMEM((CHUNK,), jnp.float32),
            pltpu.VMEM((B_SEGS,), jnp.int32),
            pltpu.VMEM((B_SEGS,), jnp.float32),
            pltpu.VMEM((B_SEGS,), jnp.int32),
            pltpu.VMEM((N_WORKERS * B_SEGS,), jnp.float32),
            pltpu.VMEM((N_WORKERS * B_SEGS,), jnp.int32),
            pltpu.VMEM((B_SEGS,), jnp.int32),
            pltpu.VMEM_SHARED((N_WORKERS * B_SEGS,), jnp.float32),
            pltpu.VMEM_SHARED((N_WORKERS * B_SEGS,), jnp.int32),
        ],
    )(values, ps32)
    return out.astype(prefix_sum.dtype)


# speedup vs baseline: 6.0730x; 6.0730x over previous
import jax
import jax.numpy as jnp
from jax import lax
from jax.experimental import pallas as pl
from jax.experimental.pallas import tpu as pltpu
from jax.experimental.pallas import tpu_sc as plsc

N_TOKENS = 32768
B_SEGS = 16
N_WORKERS = 16
CHUNK = N_TOKENS // N_WORKERS
LANES = 16


def _jagged_argmax_body(values_hbm, ps_hbm, out_hbm,
                        vals_v, ps_v, pm_v, pi_v, mm_v, mi_v, ans_v,
                        cmtab_v, citab_v, sh_max, sh_idx):
    cid = lax.axis_index("c")
    sid = lax.axis_index("s")

    @pl.when(cid == 0)
    def _():
        pltpu.sync_copy(ps_hbm, ps_v)
        lo = sid * CHUNK
        pltpu.sync_copy(values_hbm.at[pl.ds(lo, CHUNK)], vals_v)

        lane = lax.iota(jnp.int32, LANES)
        ninf = jnp.full((LANES,), -jnp.inf, jnp.float32)
        sent = jnp.full((LANES,), N_TOKENS, jnp.int32)

        ps_vec = ps_v[...]
        for k in range(B_SEGS):
            start = ps_vec[k - 1] if k > 0 else jnp.int32(0)
            end = ps_vec[k]
            a = jnp.maximum(start, lo)
            b = jnp.minimum(end, lo + CHUNK)
            va = lax.shift_right_logical(a, 4)
            vb = lax.shift_right_logical(b + (LANES - 1), 4)

            def body(j, carry, a=a, b=b, lo=lo):
                cm, ci = carry
                off = j * LANES
                v = vals_v[pl.ds(off - lo, LANES)]
                pos = off + lane
                valid = (pos >= a) & (pos < b)
                vm = jnp.where(valid, v, ninf)
                upd = vm > cm
                return jnp.where(upd, vm, cm), jnp.where(upd, pos, ci)

            cm, ci = lax.fori_loop(va, vb, body, (ninf, sent))
            cmtab_v[pl.ds(k * LANES, LANES)] = cm
            citab_v[pl.ds(k * LANES, LANES)] = ci

        pm = plsc.load_gather(cmtab_v, [lane * LANES])
        pi = plsc.load_gather(citab_v, [lane * LANES])
        for l in range(1, LANES):
            tm = plsc.load_gather(cmtab_v, [lane * LANES + l])
            ti = plsc.load_gather(citab_v, [lane * LANES + l])
            better = (tm > pm) | ((tm == pm) & (ti < pi))
            pm = jnp.where(better, tm, pm)
            pi = jnp.where(better, ti, pi)
        pm_v[...] = pm
        pi_v[...] = pi

        pltpu.sync_copy(pm_v, sh_max.at[pl.ds(sid * B_SEGS, B_SEGS)])
        pltpu.sync_copy(pi_v, sh_idx.at[pl.ds(sid * B_SEGS, B_SEGS)])
        plsc.subcore_barrier()

        @pl.when(sid == 0)
        def _():
            pltpu.sync_copy(sh_max, mm_v)
            pltpu.sync_copy(sh_idx, mi_v)
            cm = mm_v[pl.ds(0, B_SEGS)]
            ci = mi_v[pl.ds(0, B_SEGS)]
            for w in range(1, N_WORKERS):
                wm = mm_v[pl.ds(w * B_SEGS, B_SEGS)]
                wi = mi_v[pl.ds(w * B_SEGS, B_SEGS)]
                better = (wm > cm) | ((wm == cm) & (wi < ci))
                cm = jnp.where(better, wm, cm)
                ci = jnp.where(better, wi, ci)
            ans_v[...] = jnp.where(ci >= N_TOKENS, -1, ci)
            pltpu.sync_copy(ans_v, out_hbm)


@jax.jit
def kernel(values, prefix_sum):
    ps32 = prefix_sum.astype(jnp.int32)
    mesh = plsc.VectorSubcoreMesh(
        core_axis_name="c", subcore_axis_name="s", num_cores=1, num_subcores=16
    )
    out = pl.kernel(
        _jagged_argmax_body,
        out_type=jax.ShapeDtypeStruct((B_SEGS,), jnp.int32),
        mesh=mesh,
        compiler_params=pltpu.CompilerParams(needs_layout_passes=False),
        scratch_types=[
            pltpu.VMEM((CHUNK,), jnp.float32),
            pltpu.VMEM((B_SEGS,), jnp.int32),
            pltpu.VMEM((B_SEGS,), jnp.float32),
            pltpu.VMEM((B_SEGS,), jnp.int32),
            pltpu.VMEM((N_WORKERS * B_SEGS,), jnp.float32),
            pltpu.VMEM((N_WORKERS * B_SEGS,), jnp.int32),
            pltpu.VMEM((B_SEGS,), jnp.int32),
            pltpu.VMEM((B_SEGS * LANES,), jnp.float32),
            pltpu.VMEM((B_SEGS * LANES,), jnp.int32),
            pltpu.VMEM_SHARED((N_WORKERS * B_SEGS,), jnp.float32),
            pltpu.VMEM_SHARED((N_WORKERS * B_SEGS,), jnp.int32),
        ],
    )(values, ps32)
    return out.astype(prefix_sum.dtype)


# speedup vs baseline: 6.3834x; 1.0511x over previous
import jax
import jax.numpy as jnp
from jax import lax
from jax.experimental import pallas as pl
from jax.experimental.pallas import tpu as pltpu
from jax.experimental.pallas import tpu_sc as plsc

N_TOKENS = 32768
B_SEGS = 16
N_WORKERS = 16
CHUNK = N_TOKENS // N_WORKERS
LANES = 16
UNROLL = 4
PAD = UNROLL * LANES


def _jagged_argmax_body(values_hbm, ps_hbm, out_hbm,
                        vals_v, ps_v, pm_v, pi_v, mm_v, mi_v, ans_v,
                        cmtab_v, citab_v, sh_max, sh_idx, vals_sem, ps_sem):
    cid = lax.axis_index("c")
    sid = lax.axis_index("s")

    @pl.when(cid == 0)
    def _():
        lo = sid * CHUNK
        vals_cp = pltpu.async_copy(values_hbm.at[pl.ds(lo, CHUNK)],
                                   vals_v.at[pl.ds(0, CHUNK)], vals_sem)
        ps_cp = pltpu.async_copy(ps_hbm, ps_v, ps_sem)
        ps_cp.wait()

        lane = lax.iota(jnp.int32, LANES)
        ninf = jnp.full((LANES,), -jnp.inf, jnp.float32)
        sent = jnp.full((LANES,), N_TOKENS, jnp.int32)

        ps_vec = ps_v[...]
        vals_cp.wait()
        for k in range(B_SEGS):
            start = ps_vec[k - 1] if k > 0 else jnp.int32(0)
            end = ps_vec[k]
            a = jnp.maximum(start, lo)
            b = jnp.minimum(end, lo + CHUNK)
            va = lax.shift_right_logical(a, 4)
            vb = lax.shift_right_logical(b + (LANES - 1), 4)

            @pl.loop(va, vb, init_carry=(ninf, sent), step=UNROLL)
            def scan(j, carry, a=a, b=b, lo=lo):
                cm, ci = carry
                for u in range(UNROLL):
                    off = (j + u) * LANES
                    v = vals_v[pl.ds(off - lo, LANES)]
                    pos = off + lane
                    valid = (pos >= a) & (pos < b)
                    vm = jnp.where(valid, v, ninf)
                    upd = vm > cm
                    cm = jnp.where(upd, vm, cm)
                    ci = jnp.where(upd, pos, ci)
                return cm, ci

            cm, ci = scan
            cmtab_v[pl.ds(k * LANES, LANES)] = cm
            citab_v[pl.ds(k * LANES, LANES)] = ci

        pm = plsc.load_gather(cmtab_v, [lane * LANES])
        pi = plsc.load_gather(citab_v, [lane * LANES])
        for l in range(1, LANES):
            tm = plsc.load_gather(cmtab_v, [lane * LANES + l])
            ti = plsc.load_gather(citab_v, [lane * LANES + l])
            better = (tm > pm) | ((tm == pm) & (ti < pi))
            pm = jnp.where(better, tm, pm)
            pi = jnp.where(better, ti, pi)
        pm_v[...] = pm
        pi_v[...] = pi

        pltpu.sync_copy(pm_v, sh_max.at[pl.ds(sid * B_SEGS, B_SEGS)])
        pltpu.sync_copy(pi_v, sh_idx.at[pl.ds(sid * B_SEGS, B_SEGS)])
        plsc.subcore_barrier()

        @pl.when(sid == 0)
        def _():
            pltpu.sync_copy(sh_max, mm_v)
            pltpu.sync_copy(sh_idx, mi_v)
            cm = mm_v[pl.ds(0, B_SEGS)]
            ci = mi_v[pl.ds(0, B_SEGS)]
            for w in range(1, N_WORKERS):
                wm = mm_v[pl.ds(w * B_SEGS, B_SEGS)]
                wi = mi_v[pl.ds(w * B_SEGS, B_SEGS)]
                better = (wm > cm) | ((wm == cm) & (wi < ci))
                cm = jnp.where(better, wm, cm)
                ci = jnp.where(better, wi, ci)
            ans_v[...] = jnp.where(ci >= N_TOKENS, -1, ci)
            pltpu.sync_copy(ans_v, out_hbm)


@jax.jit
def kernel(values, prefix_sum):
    ps32 = prefix_sum.astype(jnp.int32)
    mesh = plsc.VectorSubcoreMesh(
        core_axis_name="c", subcore_axis_name="s", num_cores=1, num_subcores=16
    )
    out = pl.kernel(
        _jagged_argmax_body,
        out_type=jax.ShapeDtypeStruct((B_SEGS,), jnp.int32),
        mesh=mesh,
        compiler_params=pltpu.CompilerParams(needs_layout_passes=False),
        scratch_types=[
            pltpu.VMEM((CHUNK + PAD,), jnp.float32),
            pltpu.VMEM((B_SEGS,), jnp.int32),
            pltpu.VMEM((B_SEGS,), jnp.float32),
            pltpu.VMEM((B_SEGS,), jnp.int32),
            pltpu.VMEM((N_WORKERS * B_SEGS,), jnp.float32),
            pltpu.VMEM((N_WORKERS * B_SEGS,), jnp.int32),
            pltpu.VMEM((B_SEGS,), jnp.int32),
            pltpu.VMEM((B_SEGS * LANES,), jnp.float32),
            pltpu.VMEM((B_SEGS * LANES,), jnp.int32),
            pltpu.VMEM_SHARED((N_WORKERS * B_SEGS,), jnp.float32),
            pltpu.VMEM_SHARED((N_WORKERS * B_SEGS,), jnp.int32),
            pltpu.SemaphoreType.DMA,
            pltpu.SemaphoreType.DMA,
        ],
    )(values, ps32)
    return out.astype(prefix_sum.dtype)
